# baseline (speedup 1.0000x reference)
import jax
import jax.numpy as jnp
from jax.experimental import pallas as pl

TOPK = 16
EPS = 1e-6


def _encoder_tail_kernel(corr_ref, seas_ref, idx_ref, wk_ref, b_ref,
                         scale_ref, bias_ref, out_ref):
    BC, L = corr_ref.shape
    C = out_ref.shape[1]
    B = BC // C
    hi = jax.lax.Precision.HIGHEST

    c = corr_ref[...]
    iota = jax.lax.broadcasted_iota(jnp.int32, (BC, L), 1)
    sel = iota == idx_ref[:, 0:1]
    for j in range(1, TOPK):
        sel = sel | (iota == idx_ref[:, j:j + 1])

    masked = jnp.where(sel, c, 0.0)
    ff = jax.lax.dot_general(
        wk_ref[...], masked, (((0,), (0,)), ((), ())),
        preferred_element_type=jnp.float32, precision=hi)
    x = seas_ref[...] + ff + b_ref[...]

    xr = x.reshape(B, C, L)
    mean = jnp.mean(xr, axis=1, keepdims=True)
    xc = xr - mean
    var = jnp.mean(xc * xc, axis=1, keepdims=True)
    normed = xc * jax.lax.rsqrt(var + EPS)
    o = normed.reshape(BC, L) * scale_ref[...] + bias_ref[...]
    for i in range(B):
        out_ref[pl.ds(i * L, L), :] = o[i * C:(i + 1) * C, :].T


def kernel(seasonal, trend, W, b, ln_scale, ln_bias):
    B, L, C = seasonal.shape
    D = W.shape[1]
    X = jnp.fft.rfft(seasonal, axis=1)
    P = X * jnp.conj(X)
    corr = jnp.fft.irfft(P, n=L, axis=1)
    mag_t = jnp.transpose(jnp.abs(corr), (0, 2, 1))
    _, topk_idx = jax.lax.top_k(mag_t, TOPK)

    corr_t = jnp.transpose(corr, (0, 2, 1)).reshape(B * C, L)
    seas_t = jnp.transpose(seasonal, (0, 2, 1)).reshape(B * C, L)
    idx2 = topk_idx.reshape(B * C, TOPK)

    wk = jnp.kron(jnp.eye(B, dtype=jnp.float32), W)
    bt = jnp.tile(b, B).reshape(B * D, 1)
    st = jnp.tile(ln_scale, B).reshape(B * D, 1)
    bst = jnp.tile(ln_bias, B).reshape(B * D, 1)

    out2 = pl.pallas_call(
        _encoder_tail_kernel,
        out_shape=jax.ShapeDtypeStruct((B * L, D), jnp.float32),
    )(corr_t, seas_t, idx2, wk, bt, st, bst)
    return (out2.reshape(B, L, D), trend)
